# Initial kernel scaffold; baseline (speedup 1.0000x reference)
import functools

import jax
import jax.numpy as jnp
from jax import lax
from jax.experimental import pallas as pl
from jax.experimental.pallas import tpu as pltpu
from jax.experimental.pallas import tpu_sc as plsc

NUM_CORES = 2
NUM_SUBCORES = 16
NUM_WORKERS = NUM_CORES * NUM_SUBCORES
CHUNK = 512


def _sc_gather(idx_hbm, table_hbm, out_hbm, idx_v, rows_v, sem):
    b_per_w = idx_hbm.shape[0] // NUM_WORKERS
    wid = lax.axis_index("s") * NUM_CORES + lax.axis_index("c")
    base = wid * b_per_w
    pltpu.sync_copy(idx_hbm.at[pl.ds(base, b_per_w)], idx_v)

    nchunks = b_per_w // CHUNK

    def body(c, carry):
        off = c * CHUNK
        pltpu.async_copy(
            table_hbm.at[idx_v.at[pl.ds(off, CHUNK)]], rows_v, sem
        ).wait()
        pltpu.sync_copy(rows_v, out_hbm.at[pl.ds(base + off, CHUNK)])
        return carry

    lax.fori_loop(0, nchunks, body, 0)


def kernel(text, emb_table):
    batch, max_words = text.shape
    emb_dim = emb_table.shape[1]
    total = batch * max_words
    flat_idx = text.reshape(total).astype(jnp.int32)

    b_per_w = total // NUM_WORKERS
    mesh = plsc.VectorSubcoreMesh(core_axis_name="c", subcore_axis_name="s")
    call = functools.partial(
        pl.kernel,
        out_type=jax.ShapeDtypeStruct((total, emb_dim), jnp.float32),
        mesh=mesh,
        scratch_types=[
            pltpu.VMEM((b_per_w,), jnp.int32),
            pltpu.VMEM((CHUNK, emb_dim), jnp.float32),
            pltpu.SemaphoreType.DMA,
        ],
    )(_sc_gather)
    out = call(flat_idx, emb_table)
    return out.reshape(batch, max_words, emb_dim)


# speedup vs baseline: 4.7452x; 4.7452x over previous
import functools

import jax
import jax.numpy as jnp
from jax import lax
from jax.experimental import pallas as pl
from jax.experimental.pallas import tpu as pltpu
from jax.experimental.pallas import tpu_sc as plsc

NUM_CORES = 2
NUM_SUBCORES = 16
NUM_WORKERS = NUM_CORES * NUM_SUBCORES
CHUNK = 512


def _sc_gather(idx_hbm, table_hbm, out_hbm, idx_v, rows_v, sem):
    b_per_w = idx_hbm.shape[0] // NUM_WORKERS
    wid = lax.axis_index("s") * NUM_CORES + lax.axis_index("c")
    base = wid * b_per_w
    pltpu.sync_copy(idx_hbm.at[pl.ds(base, b_per_w)], idx_v)

    nchunks = b_per_w // CHUNK

    def body(c, carry):
        off = c * CHUNK
        pltpu.async_copy(
            table_hbm.at[idx_v.at[pl.ds(off, CHUNK)]], rows_v, sem
        ).wait()
        pltpu.sync_copy(rows_v, out_hbm.at[pl.ds(base + off, CHUNK)])
        return carry

    lax.fori_loop(0, nchunks, body, 0)


def kernel(text, emb_table):
    batch, max_words = text.shape
    emb_dim = emb_table.shape[1]
    total = batch * max_words
    flat_idx = text.reshape(total).astype(jnp.int32)

    b_per_w = total // NUM_WORKERS
    mesh = plsc.VectorSubcoreMesh(core_axis_name="c", subcore_axis_name="s")
    call = functools.partial(
        pl.kernel,
        out_type=jax.ShapeDtypeStruct((total, emb_dim), jnp.float32),
        mesh=mesh,
        scratch_types=[
            pltpu.VMEM((b_per_w,), jnp.int32),
            pltpu.VMEM((CHUNK, emb_dim), jnp.float32),
            pltpu.SemaphoreType.DMA,
        ],
        compiler_params=pltpu.CompilerParams(use_tc_tiling_on_sc=False),
    )(_sc_gather)
    out = call(flat_idx, emb_table)
    return out.reshape(batch, max_words, emb_dim)


# speedup vs baseline: 4.9456x; 1.0422x over previous
import functools

import jax
import jax.numpy as jnp
from jax import lax
from jax.experimental import pallas as pl
from jax.experimental.pallas import tpu as pltpu
from jax.experimental.pallas import tpu_sc as plsc

NUM_CORES = 2
NUM_SUBCORES = 16
NUM_WORKERS = NUM_CORES * NUM_SUBCORES
CHUNK = 768


def _sc_gather(idx_hbm, table_hbm, out_hbm, idx_v, rows0, rows1, gsem0, gsem1,
               wsem0, wsem1):
    b_per_w = idx_hbm.shape[0] // NUM_WORKERS
    wid = lax.axis_index("s") * NUM_CORES + lax.axis_index("c")
    base = wid * b_per_w
    pltpu.sync_copy(idx_hbm.at[pl.ds(base, b_per_w)], idx_v)

    nchunks = b_per_w // CHUNK
    bufs = (rows0, rows1)
    gsems = (gsem0, gsem1)
    wsems = (wsem0, wsem1)

    def start_gather(c, b):
        pltpu.async_copy(
            table_hbm.at[idx_v.at[pl.ds(c * CHUNK, CHUNK)]], bufs[b], gsems[b]
        )

    def wait_gather(b):
        pltpu.make_async_copy(
            table_hbm.at[idx_v.at[pl.ds(0, CHUNK)]], bufs[b], gsems[b]
        ).wait()

    def start_write(c, b):
        pltpu.async_copy(bufs[b], out_hbm.at[pl.ds(base + c * CHUNK, CHUNK)],
                         wsems[b])

    def wait_write(b):
        pltpu.make_async_copy(
            bufs[b], out_hbm.at[pl.ds(base, CHUNK)], wsems[b]
        ).wait()

    start_gather(0, 0)
    start_gather(1, 1)
    wait_gather(0)
    start_write(0, 0)

    assert nchunks % 2 == 0 and nchunks >= 4

    def body(i, carry):
        for j in range(2):
            c = 1 + i * 2 + j
            b = (1 + j) % 2
            wait_write(1 - b)
            start_gather(c + 1, 1 - b)
            wait_gather(b)
            start_write(c, b)
        return carry

    lax.fori_loop(0, (nchunks - 2) // 2, body, 0)

    wait_write(0)
    wait_gather(1)
    start_write(nchunks - 1, 1)
    wait_write(1)


def kernel(text, emb_table):
    batch, max_words = text.shape
    emb_dim = emb_table.shape[1]
    total = batch * max_words
    flat_idx = text.reshape(total).astype(jnp.int32)

    b_per_w = total // NUM_WORKERS
    mesh = plsc.VectorSubcoreMesh(core_axis_name="c", subcore_axis_name="s")
    call = functools.partial(
        pl.kernel,
        out_type=jax.ShapeDtypeStruct((total, emb_dim), jnp.float32),
        mesh=mesh,
        scratch_types=[
            pltpu.VMEM((b_per_w,), jnp.int32),
            pltpu.VMEM((CHUNK, emb_dim), jnp.float32),
            pltpu.VMEM((CHUNK, emb_dim), jnp.float32),
            pltpu.SemaphoreType.DMA,
            pltpu.SemaphoreType.DMA,
            pltpu.SemaphoreType.DMA,
            pltpu.SemaphoreType.DMA,
        ],
        compiler_params=pltpu.CompilerParams(use_tc_tiling_on_sc=False),
    )(_sc_gather)
    out = call(flat_idx, emb_table)
    return out.reshape(batch, max_words, emb_dim)


# speedup vs baseline: 10.2711x; 2.0768x over previous
import functools

import jax
import jax.numpy as jnp
from jax import lax
from jax.experimental import pallas as pl
from jax.experimental.pallas import tpu as pltpu
from jax.experimental.pallas import tpu_sc as plsc

NUM_CORES = 2
NUM_SUBCORES = 16
NUM_WORKERS = NUM_CORES * NUM_SUBCORES

ROWS_PAD = 27696
BBLK = 4096
NB = BBLK // 128
MAX_WORDS_C = 30
NBLK = 16384 // BBLK
T_TOTAL = MAX_WORDS_C * NBLK


def _sc_gather(idx_hbm, table_hbm, out_hbm, tflat, idx0, idx1, sa0, sb0, sa1,
               sb1, isem0, isem1, osem0, osem1):
    wid = lax.axis_index("s") * NUM_CORES + lax.axis_index("c")
    ehi = wid // 4
    el0 = 2 * (wid % 4)

    pltpu.sync_copy(table_hbm.at[2 * wid], tflat.at[pl.ds(0, ROWS_PAD)])
    pltpu.sync_copy(table_hbm.at[2 * wid + 1],
                    tflat.at[pl.ds(ROWS_PAD, ROWS_PAD)])

    idxb = (idx0, idx1)
    sab = (sa0, sa1)
    sbb = (sb0, sb1)
    isem = (isem0, isem1)
    osem = (osem0, osem1)

    def wb(t):
        return lax.shift_right_logical(t, 2), lax.bitwise_and(t, NBLK - 1)

    def start_idx(t, p):
        w, blk = wb(t)
        pltpu.async_copy(idx_hbm.at[w, pl.ds(blk * BBLK, BBLK)], idxb[p],
                         isem[p])

    def wait_idx(p):
        pltpu.make_async_copy(idx_hbm.at[0, pl.ds(0, BBLK)], idxb[p],
                              isem[p]).wait()

    def compute(p):
        buf = idxb[p]
        sa = sab[p]
        sb = sbb[p]

        @pl.loop(0, NB)
        def nb_body(bh):
            for j in range(8):
                iv = buf[pl.ds(bh * 128 + j * 16, 16)]
                r0 = plsc.load_gather(tflat, [iv])
                r1 = plsc.load_gather(tflat, [iv + ROWS_PAD])
                sa[bh, pl.ds(j * 16, 16)] = r0
                sb[bh, pl.ds(j * 16, 16)] = r1

    def start_out(t, p):
        w, blk = wb(t)
        pltpu.async_copy(sab[p],
                         out_hbm.at[w, ehi, pl.ds(blk * NB, NB), el0],
                         osem[p])
        pltpu.async_copy(sbb[p],
                         out_hbm.at[w, ehi, pl.ds(blk * NB, NB), el0 + 1],
                         osem[p])

    def wait_out(p):
        pltpu.make_async_copy(sab[p], out_hbm.at[0, 0, pl.ds(0, NB), 0],
                              osem[p]).wait()
        pltpu.make_async_copy(sbb[p], out_hbm.at[0, 0, pl.ds(0, NB), 0],
                              osem[p]).wait()

    start_idx(0, 0)
    start_idx(1, 1)
    for t in (0, 1):
        p = t % 2
        wait_idx(p)
        compute(p)
        start_out(t, p)
        start_idx(t + 2, p)

    def body(i, carry):
        for p in range(2):
            t = 2 + 2 * i + p
            wait_idx(p)
            wait_out(p)
            compute(p)
            start_out(t, p)
            start_idx(t + 2, p)
        return carry

    lax.fori_loop(0, (T_TOTAL - 4) // 2, body, 0)

    for t in (T_TOTAL - 2, T_TOTAL - 1):
        p = t % 2
        wait_idx(p)
        wait_out(p)
        compute(p)
        start_out(t, p)
    wait_out(0)
    wait_out(1)


def kernel(text, emb_table):
    batch, max_words = text.shape
    emb_dim = emb_table.shape[1]
    n_rows = emb_table.shape[0]

    idx_t = text.astype(jnp.int32).T
    table_t = jnp.pad(emb_table.T, ((0, 0), (0, ROWS_PAD - n_rows)))

    mesh = plsc.VectorSubcoreMesh(core_axis_name="c", subcore_axis_name="s")
    call = functools.partial(
        pl.kernel,
        out_type=jax.ShapeDtypeStruct((max_words, 8, 128, 8, 128),
                                      jnp.float32),
        mesh=mesh,
        scratch_types=[
            pltpu.VMEM((2 * ROWS_PAD,), jnp.float32),
            pltpu.VMEM((BBLK,), jnp.int32),
            pltpu.VMEM((BBLK,), jnp.int32),
            pltpu.VMEM((NB, 128), jnp.float32),
            pltpu.VMEM((NB, 128), jnp.float32),
            pltpu.VMEM((NB, 128), jnp.float32),
            pltpu.VMEM((NB, 128), jnp.float32),
            pltpu.SemaphoreType.DMA,
            pltpu.SemaphoreType.DMA,
            pltpu.SemaphoreType.DMA,
            pltpu.SemaphoreType.DMA,
        ],
        compiler_params=pltpu.CompilerParams(use_tc_tiling_on_sc=False,
                                             needs_layout_passes=False),
    )(_sc_gather)
    out5 = call(idx_t, table_t)
    return out5.transpose((2, 4, 0, 1, 3)).reshape(batch, max_words, emb_dim)


# speedup vs baseline: 12.7088x; 1.2373x over previous
import functools

import jax
import jax.numpy as jnp
from jax import lax
from jax.experimental import pallas as pl
from jax.experimental.pallas import tpu as pltpu
from jax.experimental.pallas import tpu_sc as plsc

NUM_CORES = 2
NUM_SUBCORES = 16
NUM_WORKERS = NUM_CORES * NUM_SUBCORES

ROWS_PAD = 27696
BBLK = 4096
NB = BBLK // 128
MAX_WORDS_C = 30
NBLK = 16384 // BBLK
T_TOTAL = MAX_WORDS_C * NBLK


def _sc_gather(idx_hbm, table_hbm, out_hbm, trow0, trow1, idx0, idx1, sa0, sb0,
               sa1, sb1, isem0, isem1, osem0, osem1):
    wid = lax.axis_index("s") * NUM_CORES + lax.axis_index("c")
    ehi = wid // 4
    el0 = 2 * (wid % 4)

    pltpu.sync_copy(table_hbm.at[2 * wid], trow0)
    pltpu.sync_copy(table_hbm.at[2 * wid + 1], trow1)

    idxb = (idx0, idx1)
    sab = (sa0, sa1)
    sbb = (sb0, sb1)
    isem = (isem0, isem1)
    osem = (osem0, osem1)

    def wb(t):
        return lax.shift_right_logical(t, 2), lax.bitwise_and(t, NBLK - 1)

    def start_idx(t, p):
        w, blk = wb(t)
        pltpu.async_copy(idx_hbm.at[w, pl.ds(blk * BBLK, BBLK)], idxb[p],
                         isem[p])

    def wait_idx(p):
        pltpu.make_async_copy(idx_hbm.at[0, pl.ds(0, BBLK)], idxb[p],
                              isem[p]).wait()

    def compute(p):
        buf = idxb[p]
        sa = sab[p]
        sb = sbb[p]

        @plsc.parallel_loop(0, NB, 1, unroll=2)
        def nb_body(bh):
            for j in range(8):
                iv = buf[pl.ds(bh * 128 + j * 16, 16)]
                r0 = plsc.load_gather(trow0, [iv])
                r1 = plsc.load_gather(trow1, [iv])
                sa[bh, pl.ds(j * 16, 16)] = r0
                sb[bh, pl.ds(j * 16, 16)] = r1

    def start_out(t, p):
        w, blk = wb(t)
        pltpu.async_copy(sab[p],
                         out_hbm.at[w, ehi, pl.ds(blk * NB, NB), el0],
                         osem[p])
        pltpu.async_copy(sbb[p],
                         out_hbm.at[w, ehi, pl.ds(blk * NB, NB), el0 + 1],
                         osem[p])

    def wait_out(p):
        pltpu.make_async_copy(sab[p], out_hbm.at[0, 0, pl.ds(0, NB), 0],
                              osem[p]).wait()
        pltpu.make_async_copy(sbb[p], out_hbm.at[0, 0, pl.ds(0, NB), 0],
                              osem[p]).wait()

    start_idx(0, 0)
    start_idx(1, 1)
    for t in (0, 1):
        p = t % 2
        wait_idx(p)
        compute(p)
        start_out(t, p)
        start_idx(t + 2, p)

    def body(i, carry):
        for p in range(2):
            t = 2 + 2 * i + p
            wait_idx(p)
            wait_out(p)
            compute(p)
            start_out(t, p)
            start_idx(t + 2, p)
        return carry

    lax.fori_loop(0, (T_TOTAL - 4) // 2, body, 0)

    for t in (T_TOTAL - 2, T_TOTAL - 1):
        p = t % 2
        wait_idx(p)
        wait_out(p)
        compute(p)
        start_out(t, p)
    wait_out(0)
    wait_out(1)


def kernel(text, emb_table):
    batch, max_words = text.shape
    emb_dim = emb_table.shape[1]
    n_rows = emb_table.shape[0]

    idx_t = text.astype(jnp.int32).T
    table_t = jnp.pad(emb_table.T, ((0, 0), (0, ROWS_PAD - n_rows)))

    mesh = plsc.VectorSubcoreMesh(core_axis_name="c", subcore_axis_name="s")
    call = functools.partial(
        pl.kernel,
        out_type=jax.ShapeDtypeStruct((max_words, 8, 128, 8, 128),
                                      jnp.float32),
        mesh=mesh,
        scratch_types=[
            pltpu.VMEM((ROWS_PAD,), jnp.float32),
            pltpu.VMEM((ROWS_PAD,), jnp.float32),
            pltpu.VMEM((BBLK,), jnp.int32),
            pltpu.VMEM((BBLK,), jnp.int32),
            pltpu.VMEM((NB, 128), jnp.float32),
            pltpu.VMEM((NB, 128), jnp.float32),
            pltpu.VMEM((NB, 128), jnp.float32),
            pltpu.VMEM((NB, 128), jnp.float32),
            pltpu.SemaphoreType.DMA,
            pltpu.SemaphoreType.DMA,
            pltpu.SemaphoreType.DMA,
            pltpu.SemaphoreType.DMA,
        ],
        compiler_params=pltpu.CompilerParams(use_tc_tiling_on_sc=False,
                                             needs_layout_passes=False),
    )(_sc_gather)
    out5 = call(idx_t, table_t)
    return out5.transpose((2, 4, 0, 1, 3)).reshape(batch, max_words, emb_dim)


# speedup vs baseline: 12.7428x; 1.0027x over previous
import functools

import jax
import jax.numpy as jnp
from jax import lax
from jax.experimental import pallas as pl
from jax.experimental.pallas import tpu as pltpu
from jax.experimental.pallas import tpu_sc as plsc

NUM_CORES = 2
NUM_SUBCORES = 16
NUM_WORKERS = NUM_CORES * NUM_SUBCORES

ROWS_PAD = 27696
BBLK = 4096
NB = BBLK // 128
MAX_WORDS_C = 30
NBLK = 16384 // BBLK
T_TOTAL = MAX_WORDS_C * NBLK


def _sc_gather(idx_hbm, table_hbm, out_hbm, trow0, trow1, idx0, idx1, sa0, sb0,
               sa1, sb1, isem0, isem1, osem0, osem1):
    wid = lax.axis_index("s") * NUM_CORES + lax.axis_index("c")
    ehi = wid // 4
    el0 = 2 * (wid % 4)

    pltpu.sync_copy(table_hbm.at[2 * wid], trow0)
    pltpu.sync_copy(table_hbm.at[2 * wid + 1], trow1)

    idxb = (idx0, idx1)
    sab = (sa0, sa1)
    sbb = (sb0, sb1)
    isem = (isem0, isem1)
    osem = (osem0, osem1)

    def wb(t):
        return lax.shift_right_logical(t, 2), lax.bitwise_and(t, NBLK - 1)

    def start_idx(t, p):
        w, blk = wb(t)
        pltpu.async_copy(idx_hbm.at[w, pl.ds(blk * BBLK, BBLK)], idxb[p],
                         isem[p])

    def wait_idx(p):
        pltpu.make_async_copy(idx_hbm.at[0, pl.ds(0, BBLK)], idxb[p],
                              isem[p]).wait()

    def compute(p):
        buf = idxb[p]
        sa = sab[p]
        sb = sbb[p]

        @plsc.parallel_loop(0, NB, 1, unroll=4)
        def nb_body(bh):
            for j in range(8):
                iv = buf[pl.ds(bh * 128 + j * 16, 16)]
                r0 = plsc.load_gather(trow0, [iv])
                r1 = plsc.load_gather(trow1, [iv])
                sa[bh, pl.ds(j * 16, 16)] = r0
                sb[bh, pl.ds(j * 16, 16)] = r1

    def start_out(t, p):
        w, blk = wb(t)
        pltpu.async_copy(sab[p],
                         out_hbm.at[w, ehi, pl.ds(blk * NB, NB), el0],
                         osem[p])
        pltpu.async_copy(sbb[p],
                         out_hbm.at[w, ehi, pl.ds(blk * NB, NB), el0 + 1],
                         osem[p])

    def wait_out(p):
        pltpu.make_async_copy(sab[p], out_hbm.at[0, 0, pl.ds(0, NB), 0],
                              osem[p]).wait()
        pltpu.make_async_copy(sbb[p], out_hbm.at[0, 0, pl.ds(0, NB), 0],
                              osem[p]).wait()

    start_idx(0, 0)
    start_idx(1, 1)
    for t in (0, 1):
        p = t % 2
        wait_idx(p)
        compute(p)
        start_out(t, p)
        start_idx(t + 2, p)

    def body(i, carry):
        for p in range(2):
            t = 2 + 2 * i + p
            wait_idx(p)
            wait_out(p)
            compute(p)
            start_out(t, p)
            start_idx(t + 2, p)
        return carry

    lax.fori_loop(0, (T_TOTAL - 4) // 2, body, 0)

    for t in (T_TOTAL - 2, T_TOTAL - 1):
        p = t % 2
        wait_idx(p)
        wait_out(p)
        compute(p)
        start_out(t, p)
    wait_out(0)
    wait_out(1)


def kernel(text, emb_table):
    batch, max_words = text.shape
    emb_dim = emb_table.shape[1]
    n_rows = emb_table.shape[0]

    idx_t = text.astype(jnp.int32).T
    table_t = jnp.pad(emb_table.T, ((0, 0), (0, ROWS_PAD - n_rows)))

    mesh = plsc.VectorSubcoreMesh(core_axis_name="c", subcore_axis_name="s")
    call = functools.partial(
        pl.kernel,
        out_type=jax.ShapeDtypeStruct((max_words, 8, 128, 8, 128),
                                      jnp.float32),
        mesh=mesh,
        scratch_types=[
            pltpu.VMEM((ROWS_PAD,), jnp.float32),
            pltpu.VMEM((ROWS_PAD,), jnp.float32),
            pltpu.VMEM((BBLK,), jnp.int32),
            pltpu.VMEM((BBLK,), jnp.int32),
            pltpu.VMEM((NB, 128), jnp.float32),
            pltpu.VMEM((NB, 128), jnp.float32),
            pltpu.VMEM((NB, 128), jnp.float32),
            pltpu.VMEM((NB, 128), jnp.float32),
            pltpu.SemaphoreType.DMA,
            pltpu.SemaphoreType.DMA,
            pltpu.SemaphoreType.DMA,
            pltpu.SemaphoreType.DMA,
        ],
        compiler_params=pltpu.CompilerParams(use_tc_tiling_on_sc=False,
                                             needs_layout_passes=False),
    )(_sc_gather)
    out5 = call(idx_t, table_t)
    return out5.transpose((2, 4, 0, 1, 3)).reshape(batch, max_words, emb_dim)
